# Initial kernel scaffold; baseline (speedup 1.0000x reference)
import jax
import jax.numpy as jnp
from jax.experimental import pallas as pl


def kernel(embedding, inputs):
    raise NotImplementedError("write your pallas kernel here")


# speedup vs baseline: 3.3245x; 3.3245x over previous
import functools

import jax
import jax.numpy as jnp
from jax import lax
from jax.experimental import pallas as pl
from jax.experimental.pallas import tpu as pltpu
from jax.experimental.pallas import tpu_sc as plsc

NC = 2
NS = 16
NW = NC * NS
CHUNK = 128
NBUF = 2


def _build(n_chunks, features):
    mesh = plsc.VectorSubcoreMesh(core_axis_name="c", subcore_axis_name="s")
    b_per_w = n_chunks * CHUNK

    @functools.partial(
        pl.kernel,
        mesh=mesh,
        out_type=jax.ShapeDtypeStruct((NW * b_per_w, features), jnp.float32),
        scratch_types=[
            pltpu.VMEM((n_chunks, CHUNK), jnp.int32),
            pltpu.VMEM((CHUNK, features), jnp.float32),
            pltpu.VMEM((CHUNK, features), jnp.float32),
            pltpu.SemaphoreType.DMA,
        ],
    )
    def emb_kernel(table_hbm, idx_hbm, out_hbm, idx_v, rows0, rows1, gsem):
        wid = lax.axis_index("s") * NC + lax.axis_index("c")
        base = wid * b_per_w
        rows = (rows0, rows1)
        pltpu.sync_copy(idx_hbm.at[wid], idx_v)
        pltpu.async_copy(table_hbm.at[idx_v.at[0]], rows[0], gsem)

        def outer(m, carry):
            for b in range(NBUF):
                g = m * NBUF + b

                @pl.when(g + 1 < n_chunks)
                def _():
                    pltpu.async_copy(
                        table_hbm.at[idx_v.at[g + 1]], rows[(b + 1) % NBUF], gsem
                    )

                pltpu.make_async_copy(
                    table_hbm.at[idx_v.at[g]], rows[b], gsem
                ).wait()
                pltpu.sync_copy(
                    rows[b], out_hbm.at[pl.ds(base + g * CHUNK, CHUNK)]
                )
            return carry

        lax.fori_loop(0, n_chunks // NBUF, outer, 0)

    return emb_kernel


def kernel(embedding, inputs):
    batch, hist = inputs.shape
    features = embedding.shape[1]
    total = batch * hist
    n_chunks = total // (NW * CHUNK)
    idx = inputs.reshape(NW, n_chunks, CHUNK)
    out = _build(n_chunks, features)(embedding, idx)
    return out.reshape(batch, hist, features)


# speedup vs baseline: 3.3586x; 1.0102x over previous
import functools

import jax
import jax.numpy as jnp
from jax import lax
from jax.experimental import pallas as pl
from jax.experimental.pallas import tpu as pltpu
from jax.experimental.pallas import tpu_sc as plsc

NC = 2
NS = 16
NW = NC * NS
CHUNK = 128
NBUF = 5
DEPTH = 3


def _build(n_chunks, features):
    mesh = plsc.VectorSubcoreMesh(core_axis_name="c", subcore_axis_name="s")
    b_per_w = n_chunks * CHUNK

    @functools.partial(
        pl.kernel,
        mesh=mesh,
        out_type=jax.ShapeDtypeStruct((NW * b_per_w, features), jnp.float32),
        scratch_types=[
            pltpu.VMEM((n_chunks, CHUNK), jnp.int32),
            pltpu.VMEM((NBUF, CHUNK, features), jnp.float32),
            pltpu.SemaphoreType.DMA,
            pltpu.SemaphoreType.DMA,
        ],
    )
    def emb_kernel(table_hbm, idx_hbm, out_hbm, idx_v, rows_v, gsem, wsem):
        wid = lax.axis_index("s") * NC + lax.axis_index("c")
        base = wid * b_per_w
        rows = tuple(rows_v.at[b] for b in range(NBUF))
        pltpu.sync_copy(idx_hbm.at[wid], idx_v)
        for g in range(DEPTH):
            pltpu.async_copy(table_hbm.at[idx_v.at[g]], rows[g], gsem)

        def outer(m, carry):
            for b in range(NBUF):
                g = m * NBUF + b
                pltpu.make_async_copy(
                    table_hbm.at[idx_v.at[g]], rows[b], gsem
                ).wait()
                pltpu.async_copy(
                    rows[b], out_hbm.at[pl.ds(base + g * CHUNK, CHUNK)], wsem
                )

                @pl.when(g + DEPTH < n_chunks)
                def _():
                    @pl.when(g + DEPTH >= NBUF)
                    def _():
                        pltpu.make_async_copy(
                            rows[(b + DEPTH) % NBUF],
                            out_hbm.at[pl.ds(base + (g + DEPTH - NBUF) * CHUNK, CHUNK)],
                            wsem,
                        ).wait()

                    pltpu.async_copy(
                        table_hbm.at[idx_v.at[g + DEPTH]],
                        rows[(b + DEPTH) % NBUF],
                        gsem,
                    )
            return carry

        lax.fori_loop(0, n_chunks // NBUF, outer, 0)
        for b in range(NBUF):
            pltpu.make_async_copy(
                rows[b], out_hbm.at[pl.ds(base, CHUNK)], wsem
            ).wait()

    return emb_kernel


def kernel(embedding, inputs):
    batch, hist = inputs.shape
    features = embedding.shape[1]
    total = batch * hist
    n_chunks = total // (NW * CHUNK)
    idx = inputs.reshape(NW, n_chunks, CHUNK)
    out = _build(n_chunks, features)(embedding, idx)
    return out.reshape(batch, hist, features)


# speedup vs baseline: 5.9024x; 1.7574x over previous
import functools

import jax
import jax.numpy as jnp
from jax import lax
from jax.experimental import pallas as pl
from jax.experimental.pallas import tpu as pltpu
from jax.experimental.pallas import tpu_sc as plsc

NC = 2
NS = 16
NW = NC * NS
NBUF = 4
DEPTH = 3


def _build(batch, hist, features):
    mesh = plsc.VectorSubcoreMesh(core_axis_name="c", subcore_axis_name="s")
    e_per_w = batch // NW

    @functools.partial(
        pl.kernel,
        mesh=mesh,
        out_type=jax.ShapeDtypeStruct((batch, hist, features), jnp.float32),
        scratch_types=[
            pltpu.VMEM((e_per_w, hist), jnp.int32),
            pltpu.VMEM((NBUF, hist, features), jnp.float32),
            pltpu.SemaphoreType.DMA,
            pltpu.SemaphoreType.DMA,
        ],
    )
    def emb_kernel(table_hbm, idx_hbm, out_hbm, idx_v, rows_v, gsem, wsem):
        wid = lax.axis_index("s") * NC + lax.axis_index("c")
        base = wid * e_per_w
        rows = tuple(rows_v.at[b] for b in range(NBUF))
        pltpu.sync_copy(idx_hbm.at[wid], idx_v)
        for e in range(DEPTH):
            pltpu.async_copy(table_hbm.at[idx_v.at[e]], rows[e], gsem)

        def outer(m, carry):
            for b in range(NBUF):
                e = m * NBUF + b
                pltpu.make_async_copy(
                    table_hbm.at[idx_v.at[e]], rows[b], gsem
                ).wait()
                pltpu.async_copy(rows[b], out_hbm.at[base + e], wsem)

                @pl.when(e + DEPTH < e_per_w)
                def _():
                    @pl.when(e + DEPTH >= NBUF)
                    def _():
                        pltpu.make_async_copy(
                            rows[(b + DEPTH) % NBUF],
                            out_hbm.at[base + e],
                            wsem,
                        ).wait()

                    pltpu.async_copy(
                        table_hbm.at[idx_v.at[e + DEPTH]],
                        rows[(b + DEPTH) % NBUF],
                        gsem,
                    )
            return carry

        lax.fori_loop(0, e_per_w // NBUF, outer, 0)
        for b in range(NBUF):
            pltpu.make_async_copy(rows[b], out_hbm.at[base], wsem).wait()

    return emb_kernel


def kernel(embedding, inputs):
    batch, hist = inputs.shape
    features = embedding.shape[1]
    idx = inputs.reshape(NW, batch // NW, hist)
    return _build(batch, hist, features)(embedding, idx)


# speedup vs baseline: 5.9114x; 1.0015x over previous
import functools

import jax
import jax.numpy as jnp
from jax import lax
from jax.experimental import pallas as pl
from jax.experimental.pallas import tpu as pltpu
from jax.experimental.pallas import tpu_sc as plsc

NC = 2
NS = 16
NW = NC * NS
NBUF = 4
DEPTH = 3


def _build(batch, hist, features):
    mesh = plsc.VectorSubcoreMesh(core_axis_name="c", subcore_axis_name="s")
    e_per_w = batch // NW

    @functools.partial(
        pl.kernel,
        mesh=mesh,
        out_type=jax.ShapeDtypeStruct((batch, hist, features), jnp.float32),
        scratch_types=[
            pltpu.VMEM((e_per_w, hist), jnp.int32),
            pltpu.VMEM((NBUF, hist, features), jnp.float32),
            pltpu.SemaphoreType.DMA,
            pltpu.SemaphoreType.DMA,
        ],
        compiler_params=pltpu.CompilerParams(use_tc_tiling_on_sc=True),
    )
    def emb_kernel(table_hbm, idx_hbm, out_hbm, idx_v, rows_v, gsem, wsem):
        wid = lax.axis_index("s") * NC + lax.axis_index("c")
        base = wid * e_per_w
        rows = tuple(rows_v.at[b] for b in range(NBUF))
        pltpu.sync_copy(idx_hbm.at[wid], idx_v)
        for e in range(DEPTH):
            pltpu.async_copy(table_hbm.at[idx_v.at[e]], rows[e], gsem)

        def outer(m, carry):
            for b in range(NBUF):
                e = m * NBUF + b
                pltpu.make_async_copy(
                    table_hbm.at[idx_v.at[e]], rows[b], gsem
                ).wait()
                pltpu.async_copy(rows[b], out_hbm.at[base + e], wsem)

                @pl.when(e + DEPTH < e_per_w)
                def _():
                    @pl.when(e + DEPTH >= NBUF)
                    def _():
                        pltpu.make_async_copy(
                            rows[(b + DEPTH) % NBUF],
                            out_hbm.at[base + e],
                            wsem,
                        ).wait()

                    pltpu.async_copy(
                        table_hbm.at[idx_v.at[e + DEPTH]],
                        rows[(b + DEPTH) % NBUF],
                        gsem,
                    )
            return carry

        lax.fori_loop(0, e_per_w // NBUF, outer, 0)
        for b in range(NBUF):
            pltpu.make_async_copy(rows[b], out_hbm.at[base], wsem).wait()

    return emb_kernel


def kernel(embedding, inputs):
    batch, hist = inputs.shape
    features = embedding.shape[1]
    idx = inputs.reshape(NW, batch // NW, hist)
    return _build(batch, hist, features)(embedding, idx)


# speedup vs baseline: 5.9136x; 1.0004x over previous
import functools

import jax
import jax.numpy as jnp
from jax import lax
from jax.experimental import pallas as pl
from jax.experimental.pallas import tpu as pltpu
from jax.experimental.pallas import tpu_sc as plsc

NC = 2
NS = 16
NW = NC * NS
NBUF = 4
DEPTH = 3


def _build(batch, hist, features):
    mesh = plsc.VectorSubcoreMesh(core_axis_name="c", subcore_axis_name="s")
    e_per_w = batch // NW

    @functools.partial(
        pl.kernel,
        mesh=mesh,
        out_type=jax.ShapeDtypeStruct((batch, hist, features), jnp.float32),
        scratch_types=[
            pltpu.VMEM((e_per_w, hist), jnp.int32),
            pltpu.VMEM((NBUF, hist, features), jnp.float32),
            pltpu.SemaphoreType.DMA,
            pltpu.SemaphoreType.DMA,
        ],
        compiler_params=pltpu.CompilerParams(
            use_tc_tiling_on_sc=True, needs_layout_passes=True
        ),
    )
    def emb_kernel(table_hbm, idx_hbm, out_hbm, idx_v, rows_v, gsem, wsem):
        wid = lax.axis_index("s") * NC + lax.axis_index("c")
        base = wid * e_per_w
        rows = tuple(rows_v.at[b] for b in range(NBUF))
        pltpu.sync_copy(idx_hbm.at[wid], idx_v)
        for e in range(DEPTH):
            pltpu.async_copy(table_hbm.at[idx_v.at[e]], rows[e], gsem)

        def outer(m, carry):
            for b in range(NBUF):
                e = m * NBUF + b
                pltpu.make_async_copy(
                    table_hbm.at[idx_v.at[e]], rows[b], gsem
                ).wait()
                pltpu.async_copy(rows[b], out_hbm.at[base + e], wsem)

                @pl.when(e + DEPTH < e_per_w)
                def _():
                    @pl.when(e + DEPTH >= NBUF)
                    def _():
                        pltpu.make_async_copy(
                            rows[(b + DEPTH) % NBUF],
                            out_hbm.at[base + e],
                            wsem,
                        ).wait()

                    pltpu.async_copy(
                        table_hbm.at[idx_v.at[e + DEPTH]],
                        rows[(b + DEPTH) % NBUF],
                        gsem,
                    )
            return carry

        lax.fori_loop(0, e_per_w // NBUF, outer, 0)
        for b in range(NBUF):
            pltpu.make_async_copy(rows[b], out_hbm.at[base], wsem).wait()

    return emb_kernel


def kernel(embedding, inputs):
    batch, hist = inputs.shape
    features = embedding.shape[1]
    idx = inputs.reshape(NW, batch // NW, hist)
    return _build(batch, hist, features)(embedding, idx)
